# Initial kernel scaffold; baseline (speedup 1.0000x reference)
import jax
import jax.numpy as jnp
from jax.experimental import pallas as pl


def kernel(x, edge_index, batch, edge_attr, params):
    raise NotImplementedError("write your pallas kernel here")


# speedup vs baseline: 15.9135x; 15.9135x over previous
import functools

import jax
import jax.numpy as jnp
from jax import lax
from jax.experimental import pallas as pl
from jax.experimental.pallas import tpu as pltpu
from jax.experimental.pallas import tpu_sc as plsc

_N = 10000
_E = 160000
_H = 256
_HEADS = 8
_DH = 32
_G = 16
_L = 4
_NB = 1000
_EB = 2000
_CH = 128
_NCH = _E // _CH
_NW = 32
_RPT = _N // 16
_SCALE = 1.0 / (32.0 ** 0.5)
_f32 = jnp.float32


def _headmat():
    r = lax.broadcasted_iota(jnp.int32, (_H, _HEADS), 0) // _DH
    c = lax.broadcasted_iota(jnp.int32, (_H, _HEADS), 1)
    return (r == c).astype(_f32)


def _headmat_t():
    r = lax.broadcasted_iota(jnp.int32, (_HEADS, _H), 0)
    c = lax.broadcasted_iota(jnp.int32, (_HEADS, _H), 1) // _DH
    return (r == c).astype(_f32)


def _spadmat():
    r = lax.broadcasted_iota(jnp.int32, (16, _H), 0)
    c = lax.broadcasted_iota(jnp.int32, (16, _H), 1) // _DH
    return ((r == c) & (r < _HEADS)).astype(_f32)


def _padmat128():
    r = lax.broadcasted_iota(jnp.int32, (_HEADS, 128), 0)
    c = lax.broadcasted_iota(jnp.int32, (_HEADS, 128), 1)
    return (r == c).astype(_f32)


def _spadmat128():
    r = lax.broadcasted_iota(jnp.int32, (128, _H), 0)
    c = lax.broadcasted_iota(jnp.int32, (128, _H), 1) // _DH
    return ((r == c) & (r < _HEADS)).astype(_f32)


_sc_mesh = plsc.VectorSubcoreMesh(core_axis_name="c", subcore_axis_name="s")


@functools.partial(
    pl.kernel,
    mesh=_sc_mesh,
    out_type=[
        jax.ShapeDtypeStruct((_E, _H), _f32),
        jax.ShapeDtypeStruct((_E, _H), _f32),
        jax.ShapeDtypeStruct((_E, _H), _f32),
    ],
    scratch_types=[
        pltpu.VMEM((_CH,), jnp.int32),
        pltpu.VMEM((_CH,), jnp.int32),
        pltpu.VMEM((_CH, _H), _f32),
        pltpu.VMEM((_CH, _H), _f32),
        pltpu.VMEM((_CH, _H), _f32),
        pltpu.SemaphoreType.DMA,
        pltpu.SemaphoreType.DMA,
        pltpu.SemaphoreType.DMA,
    ],
)
def _sc_gather(q_hbm, k_hbm, v_hbm, src_hbm, dst_hbm,
               qg_out, kg_out, vg_out,
               idx_s, idx_d, bufq, bufk, bufv, semq, semk, semv):
    c = lax.axis_index("c")
    s = lax.axis_index("s")
    wid = s * 2 + c

    def body(j, carry):
        cid = j * _NW + wid

        @pl.when(cid < _NCH)
        def _():
            base = cid * _CH
            pltpu.sync_copy(dst_hbm.at[pl.ds(base, _CH)], idx_d)
            pltpu.sync_copy(src_hbm.at[pl.ds(base, _CH)], idx_s)
            cq = pltpu.async_copy(q_hbm.at[idx_d], bufq, semq)
            ck = pltpu.async_copy(k_hbm.at[idx_s], bufk, semk)
            cv = pltpu.async_copy(v_hbm.at[idx_s], bufv, semv)
            cq.wait()
            ck.wait()
            cv.wait()
            pltpu.sync_copy(bufq, qg_out.at[pl.ds(base, _CH)])
            pltpu.sync_copy(bufk, kg_out.at[pl.ds(base, _CH)])
            pltpu.sync_copy(bufv, vg_out.at[pl.ds(base, _CH)])

        return carry

    lax.fori_loop(0, (_NCH + _NW - 1) // _NW, body, 0)


@functools.partial(
    pl.kernel,
    mesh=_sc_mesh,
    out_type=[jax.ShapeDtypeStruct((_N, _H), _f32)],
    scratch_types=[
        pltpu.VMEM((_CH,), jnp.int32),
        pltpu.VMEM((_CH, 128), _f32),
        pltpu.VMEM_SHARED((_N, 128), _f32),
    ],
)
def _sc_scatter_u(wv_hbm, dst_hbm, z_hbm, u_out, idx_d, bufu, shu):
    c = lax.axis_index("c")
    s = lax.axis_index("s")

    def _init(r0, nr):
        pltpu.sync_copy(z_hbm.at[pl.ds(r0, nr)], shu.at[pl.ds(r0, nr)])

    @pl.when(s < 15)
    def _():
        _init(s * 624, 624)

    @pl.when(s == 15)
    def _():
        _init(15 * 624, 640)

    plsc.subcore_barrier()

    def body(j, carry):
        cid = j * 16 + s

        @pl.when(cid < _NCH)
        def _():
            base = cid * _CH
            pltpu.sync_copy(dst_hbm.at[pl.ds(base, _CH)], idx_d)

            @pl.when(c == 0)
            def _():
                pltpu.sync_copy(wv_hbm.at[pl.ds(base, _CH), pl.ds(0, 128)],
                                bufu)

            @pl.when(c == 1)
            def _():
                pltpu.sync_copy(wv_hbm.at[pl.ds(base, _CH), pl.ds(128, 128)],
                                bufu)

            pltpu.sync_copy(bufu, shu.at[idx_d], add=True)

        return carry

    lax.fori_loop(0, (_NCH + 15) // 16, body, 0)
    plsc.subcore_barrier()

    def _wout(r0, nr):
        @pl.when(c == 0)
        def _():
            pltpu.sync_copy(shu.at[pl.ds(r0, nr)],
                            u_out.at[pl.ds(r0, nr), pl.ds(0, 128)])

        @pl.when(c == 1)
        def _():
            pltpu.sync_copy(shu.at[pl.ds(r0, nr)],
                            u_out.at[pl.ds(r0, nr), pl.ds(128, 128)])

    @pl.when(s < 15)
    def _():
        _wout(s * 624, 624)

    @pl.when(s == 15)
    def _():
        _wout(15 * 624, 640)


@functools.partial(
    pl.kernel,
    mesh=_sc_mesh,
    out_type=[
        jax.ShapeDtypeStruct((_N, 128), _f32),
        jax.ShapeDtypeStruct((_N, 128), _f32),
    ],
    scratch_types=[
        pltpu.VMEM((_CH,), jnp.int32),
        pltpu.VMEM((_CH, 128), _f32),
        pltpu.VMEM_SHARED((_N, 128), _f32),
    ],
)
def _sc_scatter_s(wp_hbm, dst_hbm, z_hbm, s0_out, s1_out, idx_d, bufw, shs):
    c = lax.axis_index("c")
    s = lax.axis_index("s")

    def _init(r0, nr):
        pltpu.sync_copy(z_hbm.at[pl.ds(r0, nr)], shs.at[pl.ds(r0, nr)])

    @pl.when(s < 15)
    def _():
        _init(s * 624, 624)

    @pl.when(s == 15)
    def _():
        _init(15 * 624, 640)

    plsc.subcore_barrier()

    def body(j, carry):
        cid = j * _NW + s * 2 + c

        @pl.when(cid < _NCH)
        def _():
            base = cid * _CH
            pltpu.sync_copy(dst_hbm.at[pl.ds(base, _CH)], idx_d)
            pltpu.sync_copy(wp_hbm.at[pl.ds(base, _CH)], bufw)
            pltpu.sync_copy(bufw, shs.at[idx_d], add=True)

        return carry

    lax.fori_loop(0, (_NCH + _NW - 1) // _NW, body, 0)
    plsc.subcore_barrier()

    def _wout(r0, nr):
        @pl.when(c == 0)
        def _():
            pltpu.sync_copy(shs.at[pl.ds(r0, nr)], s0_out.at[pl.ds(r0, nr)])

        @pl.when(c == 1)
        def _():
            pltpu.sync_copy(shs.at[pl.ds(r0, nr)], s1_out.at[pl.ds(r0, nr)])

    @pl.when(s < 15)
    def _():
        _wout(s * 624, 624)

    @pl.when(s == 15)
    def _():
        _wout(15 * 624, 640)


def _in_body(x_ref, w_ref, b_ref, o_ref):
    o_ref[...] = jnp.dot(x_ref[...], w_ref[...],
                         preferred_element_type=_f32) + b_ref[...]


def _qkv_body(h_ref, wq_ref, bq_ref, wk_ref, bk_ref, wv_ref, bv_ref,
              q_ref, k_ref, v_ref):
    hb = h_ref[...]
    q_ref[...] = jnp.dot(hb, wq_ref[...], preferred_element_type=_f32) + bq_ref[...]
    k_ref[...] = jnp.dot(hb, wk_ref[...], preferred_element_type=_f32) + bk_ref[...]
    v_ref[...] = jnp.dot(hb, wv_ref[...], preferred_element_type=_f32) + bv_ref[...]


def _logit_body(qg_ref, kg_ref, e_ref, l_ref, m_ref):
    prod = qg_ref[...] * (kg_ref[...] + e_ref[...])
    lg = jnp.dot(prod, _headmat(), precision=lax.Precision.HIGHEST,
                 preferred_element_type=_f32) * _SCALE
    l_ref[...] = lg
    m_ref[0] = jnp.max(lg, axis=0, keepdims=True)


def _w_body(l_ref, bm_ref, vg_ref, e_ref, wv_ref, wp_ref):
    cmax = jnp.max(bm_ref[:, 0, :], axis=0, keepdims=True)
    w = jnp.exp(l_ref[...] - cmax)
    vj = vg_ref[...] + e_ref[...]
    wv_ref[...] = jnp.dot(w, _headmat_t(), precision=lax.Precision.HIGHEST,
                          preferred_element_type=_f32) * vj
    wp_ref[...] = jnp.dot(w, _padmat128(), precision=lax.Precision.HIGHEST,
                          preferred_element_type=_f32)


def _post_body(h_ref, u_ref, sp0_ref, sp1_ref, wsk_ref, bsk_ref, wb_ref,
               g_ref, b_ref, o_ref):
    sb = jnp.dot(sp0_ref[...] + sp1_ref[...], _spadmat128(),
                 precision=lax.Precision.HIGHEST, preferred_element_type=_f32)
    out = jnp.where(sb > 0.0, u_ref[...] / sb, 0.0)
    xr = jnp.dot(h_ref[...], wsk_ref[...], preferred_element_type=_f32) + bsk_ref[...]
    wb = wb_ref[...]
    bl = (jnp.dot(out, wb[0:256], preferred_element_type=_f32)
          + jnp.dot(xr, wb[256:512], preferred_element_type=_f32)
          + jnp.dot(out - xr, wb[512:768], preferred_element_type=_f32))
    beta = jax.nn.sigmoid(bl)
    o2 = beta * xr + (1.0 - beta) * out
    m = jnp.mean(o2, axis=-1, keepdims=True)
    var = jnp.mean((o2 - m) ** 2, axis=-1, keepdims=True)
    hn = (o2 - m) / jnp.sqrt(var + 1e-5) * g_ref[...] + b_ref[...]
    o_ref[...] = jnp.maximum(hn, 0.0) + h_ref[...]


def _pool_body(h_ref, b_ref, sum_ref, cnt_ref):
    j = pl.program_id(0)
    oh = (b_ref[...] == lax.broadcasted_iota(jnp.int32, (1, _G), 1)).astype(_f32)
    ps = lax.dot_general(oh, h_ref[...], (((0,), (0,)), ((), ())),
                         precision=lax.Precision.HIGHEST,
                         preferred_element_type=_f32)
    ones = jnp.ones((_NB, 1), _f32)
    pc = lax.dot_general(oh, ones, (((0,), (0,)), ((), ())),
                         precision=lax.Precision.HIGHEST,
                         preferred_element_type=_f32)

    @pl.when(j == 0)
    def _():
        sum_ref[...] = ps
        cnt_ref[...] = pc

    @pl.when(j != 0)
    def _():
        sum_ref[...] = sum_ref[...] + ps
        cnt_ref[...] = cnt_ref[...] + pc


def _head_body(sum_ref, cnt_ref, w1_ref, b1_ref, g_ref, bb_ref,
               w2_ref, b2_ref, w3_ref, b3_ref, z_ref):
    sums = sum_ref[...]
    cnts = jnp.maximum(cnt_ref[...], 1.0)
    means = sums / cnts
    w1 = w1_ref[...]
    z1 = (jnp.dot(means, w1[0:256], preferred_element_type=_f32)
          + jnp.dot(sums, w1[256:512], preferred_element_type=_f32)) + b1_ref[...]
    m = jnp.mean(z1, axis=-1, keepdims=True)
    var = jnp.mean((z1 - m) ** 2, axis=-1, keepdims=True)
    z1 = (z1 - m) / jnp.sqrt(var + 1e-5) * g_ref[...] + bb_ref[...]
    z1 = jnp.maximum(z1, 0.0)
    z2 = jnp.maximum(jnp.dot(z1, w2_ref[...], preferred_element_type=_f32)
                     + b2_ref[...], 0.0)
    z_ref[...] = jnp.dot(z2, w3_ref[...], preferred_element_type=_f32) + b3_ref[...]


def kernel(x, edge_index, batch, edge_attr, params):
    p = params
    src = edge_index[0]
    dst = edge_index[1]
    ng = _N // _NB
    eg = _E // _EB

    ea = pl.pallas_call(
        _in_body,
        grid=(eg,),
        in_specs=[
            pl.BlockSpec((_EB, 16), lambda j: (j, 0)),
            pl.BlockSpec((16, _H), lambda j: (0, 0)),
            pl.BlockSpec((1, _H), lambda j: (0, 0)),
        ],
        out_specs=pl.BlockSpec((_EB, _H), lambda j: (j, 0)),
        out_shape=jax.ShapeDtypeStruct((_E, _H), _f32),
    )(edge_attr, p['W_ep'], p['b_ep'].reshape(1, _H))
    zb = jnp.zeros((1, _H), _f32)

    h = pl.pallas_call(
        _in_body,
        grid=(ng,),
        in_specs=[
            pl.BlockSpec((_NB, 256), lambda j: (j, 0)),
            pl.BlockSpec((256, _H), lambda j: (0, 0)),
            pl.BlockSpec((1, _H), lambda j: (0, 0)),
        ],
        out_specs=pl.BlockSpec((_NB, _H), lambda j: (j, 0)),
        out_shape=jax.ShapeDtypeStruct((_N, _H), _f32),
    )(x, p['W_in'], p['b_in'].reshape(1, _H))

    zerosp = jnp.zeros((_N, 128), _f32)

    for i in range(_L):
        q, k, v = pl.pallas_call(
            _qkv_body,
            grid=(ng,),
            in_specs=[
                pl.BlockSpec((_NB, _H), lambda j: (j, 0)),
                pl.BlockSpec((_H, _H), lambda j: (0, 0)),
                pl.BlockSpec((1, _H), lambda j: (0, 0)),
                pl.BlockSpec((_H, _H), lambda j: (0, 0)),
                pl.BlockSpec((1, _H), lambda j: (0, 0)),
                pl.BlockSpec((_H, _H), lambda j: (0, 0)),
                pl.BlockSpec((1, _H), lambda j: (0, 0)),
            ],
            out_specs=[pl.BlockSpec((_NB, _H), lambda j: (j, 0))] * 3,
            out_shape=[jax.ShapeDtypeStruct((_N, _H), _f32)] * 3,
        )(h, p['Wq'][i], p['bq'][i].reshape(1, _H),
          p['Wk'][i], p['bk'][i].reshape(1, _H),
          p['Wv'][i], p['bv'][i].reshape(1, _H))

        e = pl.pallas_call(
            _in_body,
            grid=(eg,),
            in_specs=[
                pl.BlockSpec((_EB, _H), lambda j: (j, 0)),
                pl.BlockSpec((_H, _H), lambda j: (0, 0)),
                pl.BlockSpec((1, _H), lambda j: (0, 0)),
            ],
            out_specs=pl.BlockSpec((_EB, _H), lambda j: (j, 0)),
            out_shape=jax.ShapeDtypeStruct((_E, _H), _f32),
        )(ea, p['We'][i], zb)

        qg, kg, vg = _sc_gather(q, k, v, src, dst)

        lg, bmax = pl.pallas_call(
            _logit_body,
            grid=(eg,),
            in_specs=[pl.BlockSpec((_EB, _H), lambda j: (j, 0))] * 3,
            out_specs=[
                pl.BlockSpec((_EB, _HEADS), lambda j: (j, 0)),
                pl.BlockSpec((1, 1, _HEADS), lambda j: (j, 0, 0)),
            ],
            out_shape=[
                jax.ShapeDtypeStruct((_E, _HEADS), _f32),
                jax.ShapeDtypeStruct((eg, 1, _HEADS), _f32),
            ],
        )(qg, kg, e)

        wv, wp = pl.pallas_call(
            _w_body,
            grid=(eg,),
            in_specs=[
                pl.BlockSpec((_EB, _HEADS), lambda j: (j, 0)),
                pl.BlockSpec((eg, 1, _HEADS), lambda j: (0, 0, 0)),
                pl.BlockSpec((_EB, _H), lambda j: (j, 0)),
                pl.BlockSpec((_EB, _H), lambda j: (j, 0)),
            ],
            out_specs=[
                pl.BlockSpec((_EB, _H), lambda j: (j, 0)),
                pl.BlockSpec((_EB, 128), lambda j: (j, 0)),
            ],
            out_shape=[
                jax.ShapeDtypeStruct((_E, _H), _f32),
                jax.ShapeDtypeStruct((_E, 128), _f32),
            ],
        )(lg, bmax, vg, e)

        (u,) = _sc_scatter_u(wv, dst, zerosp)
        s0, s1 = _sc_scatter_s(wp, dst, zerosp)

        h = pl.pallas_call(
            _post_body,
            grid=(ng,),
            in_specs=[
                pl.BlockSpec((_NB, _H), lambda j: (j, 0)),
                pl.BlockSpec((_NB, _H), lambda j: (j, 0)),
                pl.BlockSpec((_NB, 128), lambda j: (j, 0)),
                pl.BlockSpec((_NB, 128), lambda j: (j, 0)),
                pl.BlockSpec((_H, _H), lambda j: (0, 0)),
                pl.BlockSpec((1, _H), lambda j: (0, 0)),
                pl.BlockSpec((768, 1), lambda j: (0, 0)),
                pl.BlockSpec((1, _H), lambda j: (0, 0)),
                pl.BlockSpec((1, _H), lambda j: (0, 0)),
            ],
            out_specs=pl.BlockSpec((_NB, _H), lambda j: (j, 0)),
            out_shape=jax.ShapeDtypeStruct((_N, _H), _f32),
        )(h, u, s0, s1, p['Wskip'][i], p['bskip'][i].reshape(1, _H),
          p['Wbeta'][i], p['ln_g'][i].reshape(1, _H), p['ln_b'][i].reshape(1, _H))

    sums, cnts = pl.pallas_call(
        _pool_body,
        grid=(ng,),
        in_specs=[
            pl.BlockSpec((_NB, _H), lambda j: (j, 0)),
            pl.BlockSpec((_NB, 1), lambda j: (j, 0)),
        ],
        out_specs=[
            pl.BlockSpec((_G, _H), lambda j: (0, 0)),
            pl.BlockSpec((_G, 1), lambda j: (0, 0)),
        ],
        out_shape=[
            jax.ShapeDtypeStruct((_G, _H), _f32),
            jax.ShapeDtypeStruct((_G, 1), _f32),
        ],
        compiler_params=pltpu.CompilerParams(
            dimension_semantics=("arbitrary",)),
    )(h, batch.reshape(_N, 1))

    z = pl.pallas_call(
        _head_body,
        out_shape=jax.ShapeDtypeStruct((_G, 1), _f32),
    )(sums, cnts, p['Wc1'], p['bc1'].reshape(1, _H),
      p['clg'].reshape(1, _H), p['clb'].reshape(1, _H),
      p['Wc2'], p['bc2'].reshape(1, 128),
      p['Wc3'], p['bc3'].reshape(1, 1))

    return z.reshape(_G)


# speedup vs baseline: 15.9523x; 1.0024x over previous
import functools

import jax
import jax.numpy as jnp
from jax import lax
from jax.experimental import pallas as pl
from jax.experimental.pallas import tpu as pltpu
from jax.experimental.pallas import tpu_sc as plsc

_N = 10000
_E = 160000
_H = 256
_HEADS = 8
_DH = 32
_G = 16
_L = 4
_NB = 1000
_EB = 2000
_CH = 128
_NCH = _E // _CH
_NW = 32
_RPT = _N // 16
_SCALE = 1.0 / (32.0 ** 0.5)
_f32 = jnp.float32


def _headmat():
    r = lax.broadcasted_iota(jnp.int32, (_H, _HEADS), 0) // _DH
    c = lax.broadcasted_iota(jnp.int32, (_H, _HEADS), 1)
    return (r == c).astype(_f32)


def _headmat_t():
    r = lax.broadcasted_iota(jnp.int32, (_HEADS, _H), 0)
    c = lax.broadcasted_iota(jnp.int32, (_HEADS, _H), 1) // _DH
    return (r == c).astype(_f32)


def _spadmat():
    r = lax.broadcasted_iota(jnp.int32, (16, _H), 0)
    c = lax.broadcasted_iota(jnp.int32, (16, _H), 1) // _DH
    return ((r == c) & (r < _HEADS)).astype(_f32)


def _padmat128():
    r = lax.broadcasted_iota(jnp.int32, (_HEADS, 128), 0)
    c = lax.broadcasted_iota(jnp.int32, (_HEADS, 128), 1)
    return (r == c).astype(_f32)


def _spadmat128():
    r = lax.broadcasted_iota(jnp.int32, (128, _H), 0)
    c = lax.broadcasted_iota(jnp.int32, (128, _H), 1) // _DH
    return ((r == c) & (r < _HEADS)).astype(_f32)


_sc_mesh = plsc.VectorSubcoreMesh(core_axis_name="c", subcore_axis_name="s")


_GCH = 64
_GNCH = _E // _GCH
_GPW = _GNCH // _NW
_GREM = _GNCH - _GPW * _NW
_GBODY = (_GPW + 2) // 2


@functools.partial(
    pl.kernel,
    mesh=_sc_mesh,
    out_type=[
        jax.ShapeDtypeStruct((_E, _H), _f32),
        jax.ShapeDtypeStruct((_E, _H), _f32),
        jax.ShapeDtypeStruct((_E, _H), _f32),
    ],
    scratch_types=[
        pltpu.VMEM((2 * _GCH,), jnp.int32),
        pltpu.VMEM((2 * _GCH,), jnp.int32),
        pltpu.VMEM((_GCH, _H), _f32),
        pltpu.VMEM((_GCH, _H), _f32),
        pltpu.VMEM((_GCH, _H), _f32),
        pltpu.VMEM((_GCH, _H), _f32),
        pltpu.VMEM((_GCH, _H), _f32),
        pltpu.VMEM((_GCH, _H), _f32),
        pltpu.SemaphoreType.DMA,
        pltpu.SemaphoreType.DMA,
        pltpu.SemaphoreType.DMA,
        pltpu.SemaphoreType.DMA,
    ],
)
def _sc_gather(q_hbm, k_hbm, v_hbm, src_hbm, dst_hbm,
               qg_out, kg_out, vg_out,
               idx_s, idx_d, bq0, bk0, bv0, bq1, bk1, bv1,
               semg0, semg1, semw0, semw1):
    c = lax.axis_index("c")
    s = lax.axis_index("s")
    wid = s * 2 + c
    start = wid * _GPW + jnp.minimum(wid, _GREM)

    def _drain(sem, bufs):
        for b in bufs:
            pltpu.make_async_copy(q_hbm.at[pl.ds(0, _GCH)], b, sem).wait()

    def body(j2, carry):
        cid_a = start + 2 * j2
        base2 = jnp.minimum(cid_a * _GCH, _E - 2 * _GCH)
        base_a = base2
        base_b = base2 + _GCH

        @pl.when(j2 > 0)
        def _():
            _drain(semw0, (bq0, bk0, bv0))

        pltpu.sync_copy(dst_hbm.at[pl.ds(base2, 2 * _GCH)], idx_d)
        pltpu.sync_copy(src_hbm.at[pl.ds(base2, 2 * _GCH)], idx_s)
        ga_q = pltpu.async_copy(q_hbm.at[idx_d.at[pl.ds(0, _GCH)]], bq0, semg0)
        ga_k = pltpu.async_copy(k_hbm.at[idx_s.at[pl.ds(0, _GCH)]], bk0, semg0)
        ga_v = pltpu.async_copy(v_hbm.at[idx_s.at[pl.ds(0, _GCH)]], bv0, semg0)

        @pl.when(j2 > 0)
        def _():
            _drain(semw1, (bq1, bk1, bv1))

        gb_q = pltpu.async_copy(q_hbm.at[idx_d.at[pl.ds(_GCH, _GCH)]], bq1, semg1)
        gb_k = pltpu.async_copy(k_hbm.at[idx_s.at[pl.ds(_GCH, _GCH)]], bk1, semg1)
        gb_v = pltpu.async_copy(v_hbm.at[idx_s.at[pl.ds(_GCH, _GCH)]], bv1, semg1)

        ga_q.wait()
        ga_k.wait()
        ga_v.wait()
        pltpu.async_copy(bq0, qg_out.at[pl.ds(base_a, _GCH)], semw0)
        pltpu.async_copy(bk0, kg_out.at[pl.ds(base_a, _GCH)], semw0)
        pltpu.async_copy(bv0, vg_out.at[pl.ds(base_a, _GCH)], semw0)

        gb_q.wait()
        gb_k.wait()
        gb_v.wait()
        pltpu.async_copy(bq1, qg_out.at[pl.ds(base_b, _GCH)], semw1)
        pltpu.async_copy(bk1, kg_out.at[pl.ds(base_b, _GCH)], semw1)
        pltpu.async_copy(bv1, vg_out.at[pl.ds(base_b, _GCH)], semw1)

        return carry

    lax.fori_loop(0, _GBODY, body, 0)
    _drain(semw0, (bq0, bk0, bv0))
    _drain(semw1, (bq1, bk1, bv1))


@functools.partial(
    pl.kernel,
    mesh=_sc_mesh,
    out_type=[jax.ShapeDtypeStruct((_N, _H), _f32)],
    scratch_types=[
        pltpu.VMEM((_CH,), jnp.int32),
        pltpu.VMEM((_CH, 128), _f32),
        pltpu.VMEM_SHARED((_N, 128), _f32),
    ],
)
def _sc_scatter_u(wv_hbm, dst_hbm, z_hbm, u_out, idx_d, bufu, shu):
    c = lax.axis_index("c")
    s = lax.axis_index("s")

    def _init(r0, nr):
        pltpu.sync_copy(z_hbm.at[pl.ds(r0, nr)], shu.at[pl.ds(r0, nr)])

    @pl.when(s < 15)
    def _():
        _init(s * 624, 624)

    @pl.when(s == 15)
    def _():
        _init(15 * 624, 640)

    plsc.subcore_barrier()

    def body(j, carry):
        cid = j * 16 + s

        @pl.when(cid < _NCH)
        def _():
            base = cid * _CH
            pltpu.sync_copy(dst_hbm.at[pl.ds(base, _CH)], idx_d)

            @pl.when(c == 0)
            def _():
                pltpu.sync_copy(wv_hbm.at[pl.ds(base, _CH), pl.ds(0, 128)],
                                bufu)

            @pl.when(c == 1)
            def _():
                pltpu.sync_copy(wv_hbm.at[pl.ds(base, _CH), pl.ds(128, 128)],
                                bufu)

            pltpu.sync_copy(bufu, shu.at[idx_d], add=True)

        return carry

    lax.fori_loop(0, (_NCH + 15) // 16, body, 0)
    plsc.subcore_barrier()

    def _wout(r0, nr):
        @pl.when(c == 0)
        def _():
            pltpu.sync_copy(shu.at[pl.ds(r0, nr)],
                            u_out.at[pl.ds(r0, nr), pl.ds(0, 128)])

        @pl.when(c == 1)
        def _():
            pltpu.sync_copy(shu.at[pl.ds(r0, nr)],
                            u_out.at[pl.ds(r0, nr), pl.ds(128, 128)])

    @pl.when(s < 15)
    def _():
        _wout(s * 624, 624)

    @pl.when(s == 15)
    def _():
        _wout(15 * 624, 640)


@functools.partial(
    pl.kernel,
    mesh=_sc_mesh,
    out_type=[
        jax.ShapeDtypeStruct((_N, 128), _f32),
        jax.ShapeDtypeStruct((_N, 128), _f32),
    ],
    scratch_types=[
        pltpu.VMEM((_CH,), jnp.int32),
        pltpu.VMEM((_CH, 128), _f32),
        pltpu.VMEM_SHARED((_N, 128), _f32),
    ],
)
def _sc_scatter_s(wp_hbm, dst_hbm, z_hbm, s0_out, s1_out, idx_d, bufw, shs):
    c = lax.axis_index("c")
    s = lax.axis_index("s")

    def _init(r0, nr):
        pltpu.sync_copy(z_hbm.at[pl.ds(r0, nr)], shs.at[pl.ds(r0, nr)])

    @pl.when(s < 15)
    def _():
        _init(s * 624, 624)

    @pl.when(s == 15)
    def _():
        _init(15 * 624, 640)

    plsc.subcore_barrier()

    def body(j, carry):
        cid = j * _NW + s * 2 + c

        @pl.when(cid < _NCH)
        def _():
            base = cid * _CH
            pltpu.sync_copy(dst_hbm.at[pl.ds(base, _CH)], idx_d)
            pltpu.sync_copy(wp_hbm.at[pl.ds(base, _CH)], bufw)
            pltpu.sync_copy(bufw, shs.at[idx_d], add=True)

        return carry

    lax.fori_loop(0, (_NCH + _NW - 1) // _NW, body, 0)
    plsc.subcore_barrier()

    def _wout(r0, nr):
        @pl.when(c == 0)
        def _():
            pltpu.sync_copy(shs.at[pl.ds(r0, nr)], s0_out.at[pl.ds(r0, nr)])

        @pl.when(c == 1)
        def _():
            pltpu.sync_copy(shs.at[pl.ds(r0, nr)], s1_out.at[pl.ds(r0, nr)])

    @pl.when(s < 15)
    def _():
        _wout(s * 624, 624)

    @pl.when(s == 15)
    def _():
        _wout(15 * 624, 640)


def _in_body(x_ref, w_ref, b_ref, o_ref):
    o_ref[...] = jnp.dot(x_ref[...], w_ref[...],
                         preferred_element_type=_f32) + b_ref[...]


def _qkv_body(h_ref, wq_ref, bq_ref, wk_ref, bk_ref, wv_ref, bv_ref,
              q_ref, k_ref, v_ref):
    hb = h_ref[...]
    q_ref[...] = jnp.dot(hb, wq_ref[...], preferred_element_type=_f32) + bq_ref[...]
    k_ref[...] = jnp.dot(hb, wk_ref[...], preferred_element_type=_f32) + bk_ref[...]
    v_ref[...] = jnp.dot(hb, wv_ref[...], preferred_element_type=_f32) + bv_ref[...]


def _logit_body(qg_ref, kg_ref, e_ref, l_ref, m_ref):
    prod = qg_ref[...] * (kg_ref[...] + e_ref[...])
    lg = jnp.dot(prod, _headmat(), precision=lax.Precision.HIGHEST,
                 preferred_element_type=_f32) * _SCALE
    l_ref[...] = lg
    m_ref[0] = jnp.max(lg, axis=0, keepdims=True)


def _w_body(l_ref, bm_ref, vg_ref, e_ref, wv_ref, wp_ref):
    cmax = jnp.max(bm_ref[:, 0, :], axis=0, keepdims=True)
    w = jnp.exp(l_ref[...] - cmax)
    vj = vg_ref[...] + e_ref[...]
    wv_ref[...] = jnp.dot(w, _headmat_t(), precision=lax.Precision.HIGHEST,
                          preferred_element_type=_f32) * vj
    wp_ref[...] = jnp.dot(w, _padmat128(), precision=lax.Precision.HIGHEST,
                          preferred_element_type=_f32)


def _post_body(h_ref, u_ref, sp0_ref, sp1_ref, wsk_ref, bsk_ref, wb_ref,
               g_ref, b_ref, o_ref):
    sb = jnp.dot(sp0_ref[...] + sp1_ref[...], _spadmat128(),
                 precision=lax.Precision.HIGHEST, preferred_element_type=_f32)
    out = jnp.where(sb > 0.0, u_ref[...] / sb, 0.0)
    xr = jnp.dot(h_ref[...], wsk_ref[...], preferred_element_type=_f32) + bsk_ref[...]
    wb = wb_ref[...]
    bl = (jnp.dot(out, wb[0:256], preferred_element_type=_f32)
          + jnp.dot(xr, wb[256:512], preferred_element_type=_f32)
          + jnp.dot(out - xr, wb[512:768], preferred_element_type=_f32))
    beta = jax.nn.sigmoid(bl)
    o2 = beta * xr + (1.0 - beta) * out
    m = jnp.mean(o2, axis=-1, keepdims=True)
    var = jnp.mean((o2 - m) ** 2, axis=-1, keepdims=True)
    hn = (o2 - m) / jnp.sqrt(var + 1e-5) * g_ref[...] + b_ref[...]
    o_ref[...] = jnp.maximum(hn, 0.0) + h_ref[...]


def _pool_body(h_ref, b_ref, sum_ref, cnt_ref):
    j = pl.program_id(0)
    oh = (b_ref[...] == lax.broadcasted_iota(jnp.int32, (1, _G), 1)).astype(_f32)
    ps = lax.dot_general(oh, h_ref[...], (((0,), (0,)), ((), ())),
                         precision=lax.Precision.HIGHEST,
                         preferred_element_type=_f32)
    ones = jnp.ones((_NB, 1), _f32)
    pc = lax.dot_general(oh, ones, (((0,), (0,)), ((), ())),
                         precision=lax.Precision.HIGHEST,
                         preferred_element_type=_f32)

    @pl.when(j == 0)
    def _():
        sum_ref[...] = ps
        cnt_ref[...] = pc

    @pl.when(j != 0)
    def _():
        sum_ref[...] = sum_ref[...] + ps
        cnt_ref[...] = cnt_ref[...] + pc


def _head_body(sum_ref, cnt_ref, w1_ref, b1_ref, g_ref, bb_ref,
               w2_ref, b2_ref, w3_ref, b3_ref, z_ref):
    sums = sum_ref[...]
    cnts = jnp.maximum(cnt_ref[...], 1.0)
    means = sums / cnts
    w1 = w1_ref[...]
    z1 = (jnp.dot(means, w1[0:256], preferred_element_type=_f32)
          + jnp.dot(sums, w1[256:512], preferred_element_type=_f32)) + b1_ref[...]
    m = jnp.mean(z1, axis=-1, keepdims=True)
    var = jnp.mean((z1 - m) ** 2, axis=-1, keepdims=True)
    z1 = (z1 - m) / jnp.sqrt(var + 1e-5) * g_ref[...] + bb_ref[...]
    z1 = jnp.maximum(z1, 0.0)
    z2 = jnp.maximum(jnp.dot(z1, w2_ref[...], preferred_element_type=_f32)
                     + b2_ref[...], 0.0)
    z_ref[...] = jnp.dot(z2, w3_ref[...], preferred_element_type=_f32) + b3_ref[...]


def kernel(x, edge_index, batch, edge_attr, params):
    p = params
    src = edge_index[0]
    dst = edge_index[1]
    ng = _N // _NB
    eg = _E // _EB

    ea = pl.pallas_call(
        _in_body,
        grid=(eg,),
        in_specs=[
            pl.BlockSpec((_EB, 16), lambda j: (j, 0)),
            pl.BlockSpec((16, _H), lambda j: (0, 0)),
            pl.BlockSpec((1, _H), lambda j: (0, 0)),
        ],
        out_specs=pl.BlockSpec((_EB, _H), lambda j: (j, 0)),
        out_shape=jax.ShapeDtypeStruct((_E, _H), _f32),
    )(edge_attr, p['W_ep'], p['b_ep'].reshape(1, _H))
    zb = jnp.zeros((1, _H), _f32)

    h = pl.pallas_call(
        _in_body,
        grid=(ng,),
        in_specs=[
            pl.BlockSpec((_NB, 256), lambda j: (j, 0)),
            pl.BlockSpec((256, _H), lambda j: (0, 0)),
            pl.BlockSpec((1, _H), lambda j: (0, 0)),
        ],
        out_specs=pl.BlockSpec((_NB, _H), lambda j: (j, 0)),
        out_shape=jax.ShapeDtypeStruct((_N, _H), _f32),
    )(x, p['W_in'], p['b_in'].reshape(1, _H))

    zerosp = jnp.zeros((_N, 128), _f32)

    for i in range(_L):
        q, k, v = pl.pallas_call(
            _qkv_body,
            grid=(ng,),
            in_specs=[
                pl.BlockSpec((_NB, _H), lambda j: (j, 0)),
                pl.BlockSpec((_H, _H), lambda j: (0, 0)),
                pl.BlockSpec((1, _H), lambda j: (0, 0)),
                pl.BlockSpec((_H, _H), lambda j: (0, 0)),
                pl.BlockSpec((1, _H), lambda j: (0, 0)),
                pl.BlockSpec((_H, _H), lambda j: (0, 0)),
                pl.BlockSpec((1, _H), lambda j: (0, 0)),
            ],
            out_specs=[pl.BlockSpec((_NB, _H), lambda j: (j, 0))] * 3,
            out_shape=[jax.ShapeDtypeStruct((_N, _H), _f32)] * 3,
        )(h, p['Wq'][i], p['bq'][i].reshape(1, _H),
          p['Wk'][i], p['bk'][i].reshape(1, _H),
          p['Wv'][i], p['bv'][i].reshape(1, _H))

        e = pl.pallas_call(
            _in_body,
            grid=(eg,),
            in_specs=[
                pl.BlockSpec((_EB, _H), lambda j: (j, 0)),
                pl.BlockSpec((_H, _H), lambda j: (0, 0)),
                pl.BlockSpec((1, _H), lambda j: (0, 0)),
            ],
            out_specs=pl.BlockSpec((_EB, _H), lambda j: (j, 0)),
            out_shape=jax.ShapeDtypeStruct((_E, _H), _f32),
        )(ea, p['We'][i], zb)

        qg, kg, vg = _sc_gather(q, k, v, src, dst)

        lg, bmax = pl.pallas_call(
            _logit_body,
            grid=(eg,),
            in_specs=[pl.BlockSpec((_EB, _H), lambda j: (j, 0))] * 3,
            out_specs=[
                pl.BlockSpec((_EB, _HEADS), lambda j: (j, 0)),
                pl.BlockSpec((1, 1, _HEADS), lambda j: (j, 0, 0)),
            ],
            out_shape=[
                jax.ShapeDtypeStruct((_E, _HEADS), _f32),
                jax.ShapeDtypeStruct((eg, 1, _HEADS), _f32),
            ],
        )(qg, kg, e)

        wv, wp = pl.pallas_call(
            _w_body,
            grid=(eg,),
            in_specs=[
                pl.BlockSpec((_EB, _HEADS), lambda j: (j, 0)),
                pl.BlockSpec((eg, 1, _HEADS), lambda j: (0, 0, 0)),
                pl.BlockSpec((_EB, _H), lambda j: (j, 0)),
                pl.BlockSpec((_EB, _H), lambda j: (j, 0)),
            ],
            out_specs=[
                pl.BlockSpec((_EB, _H), lambda j: (j, 0)),
                pl.BlockSpec((_EB, 128), lambda j: (j, 0)),
            ],
            out_shape=[
                jax.ShapeDtypeStruct((_E, _H), _f32),
                jax.ShapeDtypeStruct((_E, 128), _f32),
            ],
        )(lg, bmax, vg, e)

        (u,) = _sc_scatter_u(wv, dst, zerosp)
        s0, s1 = _sc_scatter_s(wp, dst, zerosp)

        h = pl.pallas_call(
            _post_body,
            grid=(ng,),
            in_specs=[
                pl.BlockSpec((_NB, _H), lambda j: (j, 0)),
                pl.BlockSpec((_NB, _H), lambda j: (j, 0)),
                pl.BlockSpec((_NB, 128), lambda j: (j, 0)),
                pl.BlockSpec((_NB, 128), lambda j: (j, 0)),
                pl.BlockSpec((_H, _H), lambda j: (0, 0)),
                pl.BlockSpec((1, _H), lambda j: (0, 0)),
                pl.BlockSpec((768, 1), lambda j: (0, 0)),
                pl.BlockSpec((1, _H), lambda j: (0, 0)),
                pl.BlockSpec((1, _H), lambda j: (0, 0)),
            ],
            out_specs=pl.BlockSpec((_NB, _H), lambda j: (j, 0)),
            out_shape=jax.ShapeDtypeStruct((_N, _H), _f32),
        )(h, u, s0, s1, p['Wskip'][i], p['bskip'][i].reshape(1, _H),
          p['Wbeta'][i], p['ln_g'][i].reshape(1, _H), p['ln_b'][i].reshape(1, _H))

    sums, cnts = pl.pallas_call(
        _pool_body,
        grid=(ng,),
        in_specs=[
            pl.BlockSpec((_NB, _H), lambda j: (j, 0)),
            pl.BlockSpec((_NB, 1), lambda j: (j, 0)),
        ],
        out_specs=[
            pl.BlockSpec((_G, _H), lambda j: (0, 0)),
            pl.BlockSpec((_G, 1), lambda j: (0, 0)),
        ],
        out_shape=[
            jax.ShapeDtypeStruct((_G, _H), _f32),
            jax.ShapeDtypeStruct((_G, 1), _f32),
        ],
        compiler_params=pltpu.CompilerParams(
            dimension_semantics=("arbitrary",)),
    )(h, batch.reshape(_N, 1))

    z = pl.pallas_call(
        _head_body,
        out_shape=jax.ShapeDtypeStruct((_G, 1), _f32),
    )(sums, cnts, p['Wc1'], p['bc1'].reshape(1, _H),
      p['clg'].reshape(1, _H), p['clb'].reshape(1, _H),
      p['Wc2'], p['bc2'].reshape(1, 128),
      p['Wc3'], p['bc3'].reshape(1, 1))

    return z.reshape(_G)


# speedup vs baseline: 16.9702x; 1.0638x over previous
import functools

import jax
import jax.numpy as jnp
from jax import lax
from jax.experimental import pallas as pl
from jax.experimental.pallas import tpu as pltpu
from jax.experimental.pallas import tpu_sc as plsc

_N = 10000
_E = 160000
_H = 256
_HEADS = 8
_DH = 32
_G = 16
_L = 4
_NB = 1000
_EB = 2000
_CH = 128
_NCH = _E // _CH
_NW = 32
_RPT = _N // 16
_SCALE = 1.0 / (32.0 ** 0.5)
_f32 = jnp.float32


def _headmat():
    r = lax.broadcasted_iota(jnp.int32, (_H, _HEADS), 0) // _DH
    c = lax.broadcasted_iota(jnp.int32, (_H, _HEADS), 1)
    return (r == c).astype(_f32)


def _headmat_t():
    r = lax.broadcasted_iota(jnp.int32, (_HEADS, _H), 0)
    c = lax.broadcasted_iota(jnp.int32, (_HEADS, _H), 1) // _DH
    return (r == c).astype(_f32)


def _spadmat():
    r = lax.broadcasted_iota(jnp.int32, (16, _H), 0)
    c = lax.broadcasted_iota(jnp.int32, (16, _H), 1) // _DH
    return ((r == c) & (r < _HEADS)).astype(_f32)


def _padmat128():
    r = lax.broadcasted_iota(jnp.int32, (_HEADS, 128), 0)
    c = lax.broadcasted_iota(jnp.int32, (_HEADS, 128), 1)
    return (r == c).astype(_f32)


def _spadmat128():
    r = lax.broadcasted_iota(jnp.int32, (128, _H), 0)
    c = lax.broadcasted_iota(jnp.int32, (128, _H), 1) // _DH
    return ((r == c) & (r < _HEADS)).astype(_f32)


_sc_mesh = plsc.VectorSubcoreMesh(core_axis_name="c", subcore_axis_name="s")


@functools.partial(
    pl.kernel,
    mesh=_sc_mesh,
    out_type=[
        jax.ShapeDtypeStruct((_E, _H), _f32),
        jax.ShapeDtypeStruct((_E, _H), _f32),
        jax.ShapeDtypeStruct((_E, _H), _f32),
    ],
    scratch_types=[
        pltpu.VMEM((_CH,), jnp.int32),
        pltpu.VMEM((_CH,), jnp.int32),
        pltpu.VMEM((_CH, _H), _f32),
        pltpu.VMEM((_CH, _H), _f32),
        pltpu.VMEM((_CH, _H), _f32),
        pltpu.SemaphoreType.DMA,
        pltpu.SemaphoreType.DMA,
        pltpu.SemaphoreType.DMA,
    ],
)
def _sc_gather(q_hbm, k_hbm, v_hbm, src_hbm, dst_hbm,
               qg_out, kg_out, vg_out,
               idx_s, idx_d, bufq, bufk, bufv, semq, semk, semv):
    c = lax.axis_index("c")
    s = lax.axis_index("s")
    wid = s * 2 + c

    def body(j, carry):
        cid = j * _NW + wid

        @pl.when(cid < _NCH)
        def _():
            base = cid * _CH
            pltpu.sync_copy(dst_hbm.at[pl.ds(base, _CH)], idx_d)
            pltpu.sync_copy(src_hbm.at[pl.ds(base, _CH)], idx_s)
            cq = pltpu.async_copy(q_hbm.at[idx_d], bufq, semq)
            ck = pltpu.async_copy(k_hbm.at[idx_s], bufk, semk)
            cv = pltpu.async_copy(v_hbm.at[idx_s], bufv, semv)
            cq.wait()
            ck.wait()
            cv.wait()
            pltpu.sync_copy(bufq, qg_out.at[pl.ds(base, _CH)])
            pltpu.sync_copy(bufk, kg_out.at[pl.ds(base, _CH)])
            pltpu.sync_copy(bufv, vg_out.at[pl.ds(base, _CH)])

        return carry

    lax.fori_loop(0, (_NCH + _NW - 1) // _NW, body, 0)


_SPW = _NCH // 16
_SREM = _NCH - _SPW * 16


@functools.partial(
    pl.kernel,
    mesh=_sc_mesh,
    out_type=[jax.ShapeDtypeStruct((_N, _H), _f32)],
    scratch_types=[
        pltpu.VMEM((_CH,), jnp.int32),
        pltpu.VMEM((_CH,), jnp.int32),
        pltpu.VMEM((_CH, 128), _f32),
        pltpu.VMEM((_CH, 128), _f32),
        pltpu.VMEM_SHARED((_N, 128), _f32),
        pltpu.SemaphoreType.DMA,
        pltpu.SemaphoreType.DMA,
    ],
)
def _sc_scatter_u(wv_hbm, dst_hbm, z_hbm, u_out,
                  idx_a, idx_b, bu0, bu1, shu, semp0, semp1):
    c = lax.axis_index("c")
    s = lax.axis_index("s")

    def _init(r0, nr):
        pltpu.sync_copy(z_hbm.at[pl.ds(r0, nr)], shu.at[pl.ds(r0, nr)])

    @pl.when(s < 15)
    def _():
        _init(s * 624, 624)

    @pl.when(s == 15)
    def _():
        _init(15 * 624, 640)

    plsc.subcore_barrier()

    start = s * _SPW + jnp.minimum(s, _SREM)
    end = start + _SPW + jnp.where(s < _SREM, 1, 0)

    def _loop(col0):
        def body(j2, carry):
            ca = start + 2 * j2
            cb = ca + 1

            @pl.when(cb < end)
            def _():
                base = ca * _CH
                la = pltpu.async_copy(
                    wv_hbm.at[pl.ds(base, _CH), pl.ds(col0, 128)], bu0, semp0)
                lb = pltpu.async_copy(
                    wv_hbm.at[pl.ds(base + _CH, _CH), pl.ds(col0, 128)],
                    bu1, semp1)
                pltpu.sync_copy(dst_hbm.at[pl.ds(base, _CH)], idx_a)
                pltpu.sync_copy(dst_hbm.at[pl.ds(base + _CH, _CH)], idx_b)
                la.wait()
                pltpu.sync_copy(bu0, shu.at[idx_a], add=True)
                lb.wait()
                pltpu.sync_copy(bu1, shu.at[idx_b], add=True)

            @pl.when(cb == end)
            def _():
                base = ca * _CH
                pltpu.sync_copy(dst_hbm.at[pl.ds(base, _CH)], idx_a)
                pltpu.sync_copy(wv_hbm.at[pl.ds(base, _CH), pl.ds(col0, 128)],
                                bu0)
                pltpu.sync_copy(bu0, shu.at[idx_a], add=True)

            return carry

        lax.fori_loop(0, (_SPW + 2) // 2, body, 0)

    @pl.when(c == 0)
    def _():
        _loop(0)

    @pl.when(c == 1)
    def _():
        _loop(128)

    plsc.subcore_barrier()

    def _wout(r0, nr):
        @pl.when(c == 0)
        def _():
            pltpu.sync_copy(shu.at[pl.ds(r0, nr)],
                            u_out.at[pl.ds(r0, nr), pl.ds(0, 128)])

        @pl.when(c == 1)
        def _():
            pltpu.sync_copy(shu.at[pl.ds(r0, nr)],
                            u_out.at[pl.ds(r0, nr), pl.ds(128, 128)])

    @pl.when(s < 15)
    def _():
        _wout(s * 624, 624)

    @pl.when(s == 15)
    def _():
        _wout(15 * 624, 640)


@functools.partial(
    pl.kernel,
    mesh=_sc_mesh,
    out_type=[
        jax.ShapeDtypeStruct((_N, 128), _f32),
        jax.ShapeDtypeStruct((_N, 128), _f32),
    ],
    scratch_types=[
        pltpu.VMEM((_CH,), jnp.int32),
        pltpu.VMEM((_CH,), jnp.int32),
        pltpu.VMEM((_CH, 128), _f32),
        pltpu.VMEM((_CH, 128), _f32),
        pltpu.VMEM_SHARED((_N, 128), _f32),
        pltpu.SemaphoreType.DMA,
        pltpu.SemaphoreType.DMA,
    ],
)
def _sc_scatter_s(wp_hbm, dst_hbm, z_hbm, s0_out, s1_out,
                  idx_a, idx_b, bw0, bw1, shs, semp0, semp1):
    c = lax.axis_index("c")
    s = lax.axis_index("s")
    wid = s * 2 + c

    def _init(r0, nr):
        pltpu.sync_copy(z_hbm.at[pl.ds(r0, nr)], shs.at[pl.ds(r0, nr)])

    @pl.when(s < 15)
    def _():
        _init(s * 624, 624)

    @pl.when(s == 15)
    def _():
        _init(15 * 624, 640)

    plsc.subcore_barrier()

    wpw = _NCH // _NW
    wrem = _NCH - wpw * _NW
    start = wid * wpw + jnp.minimum(wid, wrem)
    end = start + wpw + jnp.where(wid < wrem, 1, 0)

    def body(j2, carry):
        ca = start + 2 * j2
        cb = ca + 1

        @pl.when(cb < end)
        def _():
            base = ca * _CH
            la = pltpu.async_copy(wp_hbm.at[pl.ds(base, _CH)], bw0, semp0)
            lb = pltpu.async_copy(wp_hbm.at[pl.ds(base + _CH, _CH)], bw1, semp1)
            pltpu.sync_copy(dst_hbm.at[pl.ds(base, _CH)], idx_a)
            pltpu.sync_copy(dst_hbm.at[pl.ds(base + _CH, _CH)], idx_b)
            la.wait()
            pltpu.sync_copy(bw0, shs.at[idx_a], add=True)
            lb.wait()
            pltpu.sync_copy(bw1, shs.at[idx_b], add=True)

        @pl.when(cb == end)
        def _():
            base = ca * _CH
            pltpu.sync_copy(dst_hbm.at[pl.ds(base, _CH)], idx_a)
            pltpu.sync_copy(wp_hbm.at[pl.ds(base, _CH)], bw0)
            pltpu.sync_copy(bw0, shs.at[idx_a], add=True)

        return carry

    lax.fori_loop(0, (wpw + 2) // 2, body, 0)
    plsc.subcore_barrier()

    def _wout(r0, nr):
        @pl.when(c == 0)
        def _():
            pltpu.sync_copy(shs.at[pl.ds(r0, nr)], s0_out.at[pl.ds(r0, nr)])

        @pl.when(c == 1)
        def _():
            pltpu.sync_copy(shs.at[pl.ds(r0, nr)], s1_out.at[pl.ds(r0, nr)])

    @pl.when(s < 15)
    def _():
        _wout(s * 624, 624)

    @pl.when(s == 15)
    def _():
        _wout(15 * 624, 640)


def _in_body(x_ref, w_ref, b_ref, o_ref):
    o_ref[...] = jnp.dot(x_ref[...], w_ref[...],
                         preferred_element_type=_f32) + b_ref[...]


def _qkv_body(h_ref, wq_ref, bq_ref, wk_ref, bk_ref, wv_ref, bv_ref,
              q_ref, k_ref, v_ref):
    hb = h_ref[...]
    q_ref[...] = jnp.dot(hb, wq_ref[...], preferred_element_type=_f32) + bq_ref[...]
    k_ref[...] = jnp.dot(hb, wk_ref[...], preferred_element_type=_f32) + bk_ref[...]
    v_ref[...] = jnp.dot(hb, wv_ref[...], preferred_element_type=_f32) + bv_ref[...]


def _logit_body(qg_ref, kg_ref, e_ref, l_ref, m_ref):
    prod = qg_ref[...] * (kg_ref[...] + e_ref[...])
    lg = jnp.dot(prod, _headmat(), precision=lax.Precision.HIGHEST,
                 preferred_element_type=_f32) * _SCALE
    l_ref[...] = lg
    m_ref[0] = jnp.max(lg, axis=0, keepdims=True)


def _w_body(l_ref, bm_ref, vg_ref, e_ref, wv_ref, wp_ref):
    cmax = jnp.max(bm_ref[:, 0, :], axis=0, keepdims=True)
    w = jnp.exp(l_ref[...] - cmax)
    vj = vg_ref[...] + e_ref[...]
    wv_ref[...] = jnp.dot(w, _headmat_t(), precision=lax.Precision.HIGHEST,
                          preferred_element_type=_f32) * vj
    wp_ref[...] = jnp.dot(w, _padmat128(), precision=lax.Precision.HIGHEST,
                          preferred_element_type=_f32)


def _post_body(h_ref, u_ref, sp0_ref, sp1_ref, wsk_ref, bsk_ref, wb_ref,
               g_ref, b_ref, o_ref):
    sb = jnp.dot(sp0_ref[...] + sp1_ref[...], _spadmat128(),
                 precision=lax.Precision.HIGHEST, preferred_element_type=_f32)
    out = jnp.where(sb > 0.0, u_ref[...] / sb, 0.0)
    xr = jnp.dot(h_ref[...], wsk_ref[...], preferred_element_type=_f32) + bsk_ref[...]
    wb = wb_ref[...]
    bl = (jnp.dot(out, wb[0:256], preferred_element_type=_f32)
          + jnp.dot(xr, wb[256:512], preferred_element_type=_f32)
          + jnp.dot(out - xr, wb[512:768], preferred_element_type=_f32))
    beta = jax.nn.sigmoid(bl)
    o2 = beta * xr + (1.0 - beta) * out
    m = jnp.mean(o2, axis=-1, keepdims=True)
    var = jnp.mean((o2 - m) ** 2, axis=-1, keepdims=True)
    hn = (o2 - m) / jnp.sqrt(var + 1e-5) * g_ref[...] + b_ref[...]
    o_ref[...] = jnp.maximum(hn, 0.0) + h_ref[...]


def _pool_body(h_ref, b_ref, sum_ref, cnt_ref):
    j = pl.program_id(0)
    oh = (b_ref[...] == lax.broadcasted_iota(jnp.int32, (1, _G), 1)).astype(_f32)
    ps = lax.dot_general(oh, h_ref[...], (((0,), (0,)), ((), ())),
                         precision=lax.Precision.HIGHEST,
                         preferred_element_type=_f32)
    ones = jnp.ones((_NB, 1), _f32)
    pc = lax.dot_general(oh, ones, (((0,), (0,)), ((), ())),
                         precision=lax.Precision.HIGHEST,
                         preferred_element_type=_f32)

    @pl.when(j == 0)
    def _():
        sum_ref[...] = ps
        cnt_ref[...] = pc

    @pl.when(j != 0)
    def _():
        sum_ref[...] = sum_ref[...] + ps
        cnt_ref[...] = cnt_ref[...] + pc


def _head_body(sum_ref, cnt_ref, w1_ref, b1_ref, g_ref, bb_ref,
               w2_ref, b2_ref, w3_ref, b3_ref, z_ref):
    sums = sum_ref[...]
    cnts = jnp.maximum(cnt_ref[...], 1.0)
    means = sums / cnts
    w1 = w1_ref[...]
    z1 = (jnp.dot(means, w1[0:256], preferred_element_type=_f32)
          + jnp.dot(sums, w1[256:512], preferred_element_type=_f32)) + b1_ref[...]
    m = jnp.mean(z1, axis=-1, keepdims=True)
    var = jnp.mean((z1 - m) ** 2, axis=-1, keepdims=True)
    z1 = (z1 - m) / jnp.sqrt(var + 1e-5) * g_ref[...] + bb_ref[...]
    z1 = jnp.maximum(z1, 0.0)
    z2 = jnp.maximum(jnp.dot(z1, w2_ref[...], preferred_element_type=_f32)
                     + b2_ref[...], 0.0)
    z_ref[...] = jnp.dot(z2, w3_ref[...], preferred_element_type=_f32) + b3_ref[...]


def kernel(x, edge_index, batch, edge_attr, params):
    p = params
    src = edge_index[0]
    dst = edge_index[1]
    ng = _N // _NB
    eg = _E // _EB

    ea = pl.pallas_call(
        _in_body,
        grid=(eg,),
        in_specs=[
            pl.BlockSpec((_EB, 16), lambda j: (j, 0)),
            pl.BlockSpec((16, _H), lambda j: (0, 0)),
            pl.BlockSpec((1, _H), lambda j: (0, 0)),
        ],
        out_specs=pl.BlockSpec((_EB, _H), lambda j: (j, 0)),
        out_shape=jax.ShapeDtypeStruct((_E, _H), _f32),
    )(edge_attr, p['W_ep'], p['b_ep'].reshape(1, _H))
    zb = jnp.zeros((1, _H), _f32)

    h = pl.pallas_call(
        _in_body,
        grid=(ng,),
        in_specs=[
            pl.BlockSpec((_NB, 256), lambda j: (j, 0)),
            pl.BlockSpec((256, _H), lambda j: (0, 0)),
            pl.BlockSpec((1, _H), lambda j: (0, 0)),
        ],
        out_specs=pl.BlockSpec((_NB, _H), lambda j: (j, 0)),
        out_shape=jax.ShapeDtypeStruct((_N, _H), _f32),
    )(x, p['W_in'], p['b_in'].reshape(1, _H))

    zerosp = jnp.zeros((_N, 128), _f32)

    for i in range(_L):
        q, k, v = pl.pallas_call(
            _qkv_body,
            grid=(ng,),
            in_specs=[
                pl.BlockSpec((_NB, _H), lambda j: (j, 0)),
                pl.BlockSpec((_H, _H), lambda j: (0, 0)),
                pl.BlockSpec((1, _H), lambda j: (0, 0)),
                pl.BlockSpec((_H, _H), lambda j: (0, 0)),
                pl.BlockSpec((1, _H), lambda j: (0, 0)),
                pl.BlockSpec((_H, _H), lambda j: (0, 0)),
                pl.BlockSpec((1, _H), lambda j: (0, 0)),
            ],
            out_specs=[pl.BlockSpec((_NB, _H), lambda j: (j, 0))] * 3,
            out_shape=[jax.ShapeDtypeStruct((_N, _H), _f32)] * 3,
        )(h, p['Wq'][i], p['bq'][i].reshape(1, _H),
          p['Wk'][i], p['bk'][i].reshape(1, _H),
          p['Wv'][i], p['bv'][i].reshape(1, _H))

        e = pl.pallas_call(
            _in_body,
            grid=(eg,),
            in_specs=[
                pl.BlockSpec((_EB, _H), lambda j: (j, 0)),
                pl.BlockSpec((_H, _H), lambda j: (0, 0)),
                pl.BlockSpec((1, _H), lambda j: (0, 0)),
            ],
            out_specs=pl.BlockSpec((_EB, _H), lambda j: (j, 0)),
            out_shape=jax.ShapeDtypeStruct((_E, _H), _f32),
        )(ea, p['We'][i], zb)

        qg, kg, vg = _sc_gather(q, k, v, src, dst)

        lg, bmax = pl.pallas_call(
            _logit_body,
            grid=(eg,),
            in_specs=[pl.BlockSpec((_EB, _H), lambda j: (j, 0))] * 3,
            out_specs=[
                pl.BlockSpec((_EB, _HEADS), lambda j: (j, 0)),
                pl.BlockSpec((1, 1, _HEADS), lambda j: (j, 0, 0)),
            ],
            out_shape=[
                jax.ShapeDtypeStruct((_E, _HEADS), _f32),
                jax.ShapeDtypeStruct((eg, 1, _HEADS), _f32),
            ],
        )(qg, kg, e)

        wv, wp = pl.pallas_call(
            _w_body,
            grid=(eg,),
            in_specs=[
                pl.BlockSpec((_EB, _HEADS), lambda j: (j, 0)),
                pl.BlockSpec((eg, 1, _HEADS), lambda j: (0, 0, 0)),
                pl.BlockSpec((_EB, _H), lambda j: (j, 0)),
                pl.BlockSpec((_EB, _H), lambda j: (j, 0)),
            ],
            out_specs=[
                pl.BlockSpec((_EB, _H), lambda j: (j, 0)),
                pl.BlockSpec((_EB, 128), lambda j: (j, 0)),
            ],
            out_shape=[
                jax.ShapeDtypeStruct((_E, _H), _f32),
                jax.ShapeDtypeStruct((_E, 128), _f32),
            ],
        )(lg, bmax, vg, e)

        (u,) = _sc_scatter_u(wv, dst, zerosp)
        s0, s1 = _sc_scatter_s(wp, dst, zerosp)

        h = pl.pallas_call(
            _post_body,
            grid=(ng,),
            in_specs=[
                pl.BlockSpec((_NB, _H), lambda j: (j, 0)),
                pl.BlockSpec((_NB, _H), lambda j: (j, 0)),
                pl.BlockSpec((_NB, 128), lambda j: (j, 0)),
                pl.BlockSpec((_NB, 128), lambda j: (j, 0)),
                pl.BlockSpec((_H, _H), lambda j: (0, 0)),
                pl.BlockSpec((1, _H), lambda j: (0, 0)),
                pl.BlockSpec((768, 1), lambda j: (0, 0)),
                pl.BlockSpec((1, _H), lambda j: (0, 0)),
                pl.BlockSpec((1, _H), lambda j: (0, 0)),
            ],
            out_specs=pl.BlockSpec((_NB, _H), lambda j: (j, 0)),
            out_shape=jax.ShapeDtypeStruct((_N, _H), _f32),
        )(h, u, s0, s1, p['Wskip'][i], p['bskip'][i].reshape(1, _H),
          p['Wbeta'][i], p['ln_g'][i].reshape(1, _H), p['ln_b'][i].reshape(1, _H))

    sums, cnts = pl.pallas_call(
        _pool_body,
        grid=(ng,),
        in_specs=[
            pl.BlockSpec((_NB, _H), lambda j: (j, 0)),
            pl.BlockSpec((_NB, 1), lambda j: (j, 0)),
        ],
        out_specs=[
            pl.BlockSpec((_G, _H), lambda j: (0, 0)),
            pl.BlockSpec((_G, 1), lambda j: (0, 0)),
        ],
        out_shape=[
            jax.ShapeDtypeStruct((_G, _H), _f32),
            jax.ShapeDtypeStruct((_G, 1), _f32),
        ],
        compiler_params=pltpu.CompilerParams(
            dimension_semantics=("arbitrary",)),
    )(h, batch.reshape(_N, 1))

    z = pl.pallas_call(
        _head_body,
        out_shape=jax.ShapeDtypeStruct((_G, 1), _f32),
    )(sums, cnts, p['Wc1'], p['bc1'].reshape(1, _H),
      p['clg'].reshape(1, _H), p['clb'].reshape(1, _H),
      p['Wc2'], p['bc2'].reshape(1, 128),
      p['Wc3'], p['bc3'].reshape(1, 1))

    return z.reshape(_G)
